# baseline (speedup 1.0000x reference)
import functools

import jax
import jax.numpy as jnp
from jax import lax
from jax.experimental import pallas as pl
from jax.experimental.pallas import tpu as pltpu
from jax.experimental.pallas import tpu_sc as plsc


def _lrelu(t):
    return jax.nn.leaky_relu(t, 0.01)


_BN = 2000
_BE = 16000


def _edge_mlp_body(attr_ref, ew_w_ref, ew_b_ref, pw_ref, pb_ref, out_ref):
    ea = _lrelu(
        jnp.dot(attr_ref[...], ew_w_ref[...], preferred_element_type=jnp.float32)
        + ew_b_ref[...]
    )
    t = jnp.dot(ea, pw_ref[...], preferred_element_type=jnp.float32) + pb_ref[...]
    ew = jax.nn.sigmoid(t)
    out_ref[...] = jnp.clip(ew, 1e-6, 1.0)


def _edge_weights(edge_attr, edge_w, edge_b, proj_w, proj_b):
    E = edge_attr.shape[0]
    grid = pl.cdiv(E, _BE)
    out = pl.pallas_call(
        _edge_mlp_body,
        grid=(grid,),
        in_specs=[
            pl.BlockSpec((_BE, edge_attr.shape[1]), lambda i: (i, 0)),
            pl.BlockSpec(edge_w.shape, lambda i: (0, 0)),
            pl.BlockSpec(edge_b.shape, lambda i: (0,)),
            pl.BlockSpec(proj_w.shape, lambda i: (0, 0)),
            pl.BlockSpec(proj_b.shape, lambda i: (0,)),
        ],
        out_specs=pl.BlockSpec((_BE, 1), lambda i: (i, 0)),
        out_shape=jax.ShapeDtypeStruct((E, 1), jnp.float32),
    )(edge_attr, edge_w, edge_b, proj_w, proj_b)
    return out.reshape(E)


def _first_y_body(x_ref, nw_ref, nb_ref, w0_ref, deg_ref, y_ref):
    h0 = _lrelu(
        jnp.dot(x_ref[...], nw_ref[...], preferred_element_type=jnp.float32)
        + nb_ref[...]
    )
    dinv = (deg_ref[0, :, 0] + deg_ref[1, :, 0] + 1.0) ** -0.5
    y_ref[...] = (
        jnp.dot(h0, w0_ref[...], preferred_element_type=jnp.float32)
        * dinv[:, None]
    )


def _first_y(x, node_w, node_b, conv_w0, deg):
    N, F = x.shape
    H = conv_w0.shape[1]
    return pl.pallas_call(
        _first_y_body,
        grid=(N // _BN,),
        in_specs=[
            pl.BlockSpec((_BN, F), lambda i: (i, 0)),
            pl.BlockSpec(node_w.shape, lambda i: (0, 0)),
            pl.BlockSpec(node_b.shape, lambda i: (0,)),
            pl.BlockSpec(conv_w0.shape, lambda i: (0, 0)),
            pl.BlockSpec((2, _BN, 1), lambda i: (0, i, 0)),
        ],
        out_specs=pl.BlockSpec((_BN, H), lambda i: (i, 0)),
        out_shape=jax.ShapeDtypeStruct((N, H), jnp.float32),
    )(x, node_w, node_b, conv_w0, deg)


def _next_y_body(s_ref, y_ref, deg_ref, cb_ref, g_ref, b_ref, w_ref, out_ref):
    dinv = (deg_ref[0, :, 0] + deg_ref[1, :, 0] + 1.0) ** -0.5
    s = s_ref[0] + s_ref[1]
    h = _lrelu(dinv[:, None] * (s + y_ref[...]) + cb_ref[...]) * g_ref[...] + b_ref[...]
    out_ref[...] = (
        jnp.dot(h, w_ref[...], preferred_element_type=jnp.float32) * dinv[:, None]
    )


def _next_y(s_parts, y_prev, deg, cb, g, b, w):
    N, H = y_prev.shape
    return pl.pallas_call(
        _next_y_body,
        grid=(N // _BN,),
        in_specs=[
            pl.BlockSpec((2, _BN, H), lambda i: (0, i, 0)),
            pl.BlockSpec((_BN, H), lambda i: (i, 0)),
            pl.BlockSpec((2, _BN, 1), lambda i: (0, i, 0)),
            pl.BlockSpec(cb.shape, lambda i: (0,)),
            pl.BlockSpec(g.shape, lambda i: (0,)),
            pl.BlockSpec(b.shape, lambda i: (0,)),
            pl.BlockSpec(w.shape, lambda i: (0, 0)),
        ],
        out_specs=pl.BlockSpec((_BN, H), lambda i: (i, 0)),
        out_shape=jax.ShapeDtypeStruct((N, H), jnp.float32),
    )(s_parts, y_prev, deg, cb, g, b, w)


def _final_body(s_ref, y_ref, deg_ref, cb_ref, g_ref, b_ref, batch_ref,
                u_ref, gw_ref, gb_ref, w1_ref, b1_ref, w2_ref, b2_ref,
                out_ref, pooled_acc, cnt_acc):
    j = pl.program_id(0)
    nb = pl.num_programs(0)

    @pl.when(j == 0)
    def _init():
        pooled_acc[...] = jnp.zeros_like(pooled_acc)
        cnt_acc[...] = jnp.zeros_like(cnt_acc)

    dinv = (deg_ref[0, :, 0] + deg_ref[1, :, 0] + 1.0) ** -0.5
    s = s_ref[0] + s_ref[1]
    h = _lrelu(dinv[:, None] * (s + y_ref[...]) + cb_ref[...]) * g_ref[...] + b_ref[...]

    B = pooled_acc.shape[0]
    onehot = (
        lax.broadcasted_iota(jnp.int32, (B, h.shape[0]), 0)
        == batch_ref[:, 0][None, :]
    ).astype(jnp.float32)
    pooled_acc[...] += jnp.dot(onehot, h, preferred_element_type=jnp.float32)
    cnt_acc[...] += jnp.sum(onehot, axis=1, keepdims=True)

    @pl.when(j == nb - 1)
    def _fin():
        pooled = pooled_acc[...] / jnp.maximum(cnt_acc[...], 1.0)
        ug = _lrelu(
            jnp.dot(u_ref[...], gw_ref[...], preferred_element_type=jnp.float32)
            + gb_ref[...]
        )
        comb = jnp.concatenate([pooled, ug], axis=1)
        hid = _lrelu(
            jnp.dot(comb, w1_ref[...], preferred_element_type=jnp.float32)
            + b1_ref[...]
        )
        out_ref[...] = (
            jnp.dot(hid, w2_ref[...], preferred_element_type=jnp.float32)
            + b2_ref[...]
        )


def _final(s_parts, y_prev, deg, cb, g, b, batch, u, glob_w, glob_b,
           fin_w1, fin_b1, fin_w2, fin_b2):
    N, H = y_prev.shape
    B = u.shape[0]
    return pl.pallas_call(
        _final_body,
        grid=(N // _BN,),
        in_specs=[
            pl.BlockSpec((2, _BN, H), lambda i: (0, i, 0)),
            pl.BlockSpec((_BN, H), lambda i: (i, 0)),
            pl.BlockSpec((2, _BN, 1), lambda i: (0, i, 0)),
            pl.BlockSpec(cb.shape, lambda i: (0,)),
            pl.BlockSpec(g.shape, lambda i: (0,)),
            pl.BlockSpec(b.shape, lambda i: (0,)),
            pl.BlockSpec((_BN, 1), lambda i: (i, 0)),
            pl.BlockSpec(u.shape, lambda i: (0, 0)),
            pl.BlockSpec(glob_w.shape, lambda i: (0, 0)),
            pl.BlockSpec(glob_b.shape, lambda i: (0,)),
            pl.BlockSpec(fin_w1.shape, lambda i: (0, 0)),
            pl.BlockSpec(fin_b1.shape, lambda i: (0,)),
            pl.BlockSpec(fin_w2.shape, lambda i: (0, 0)),
            pl.BlockSpec(fin_b2.shape, lambda i: (0,)),
        ],
        out_specs=pl.BlockSpec((B, 2), lambda i: (0, 0)),
        out_shape=jax.ShapeDtypeStruct((B, 2), jnp.float32),
        scratch_shapes=[
            pltpu.VMEM((B, H), jnp.float32),
            pltpu.VMEM((B, 1), jnp.float32),
        ],
    )(s_parts, y_prev, deg, cb, g, b, batch, u, glob_w, glob_b,
      fin_w1, fin_b1, fin_w2, fin_b2)


_NC = 2
_NS = 16
_MESH = dict(core_axis_name="c", subcore_axis_name="s",
             num_cores=_NC, num_subcores=_NS)


def _deg_kernel(N, E):
    epw = E // (_NC * _NS)
    K = 1000
    nch = epw // K
    mesh = plsc.VectorSubcoreMesh(**_MESH)

    @functools.partial(
        pl.kernel,
        out_type=jax.ShapeDtypeStruct((_NC, N), jnp.float32),
        mesh=mesh,
        scratch_types=[
            pltpu.VMEM((K,), jnp.int32),
            pltpu.VMEM((K,), jnp.float32),
            pltpu.VMEM_SHARED((N,), jnp.float32),
        ],
    )
    def k(col_hbm, ew_hbm, zeros_hbm, out_hbm, colbuf, ewbuf, acc):
        c = lax.axis_index("c")
        s = lax.axis_index("s")

        @pl.when(s == 0)
        def _zero():
            pltpu.sync_copy(zeros_hbm, acc)

        plsc.subcore_barrier()
        base = (c * _NS + s) * epw

        def body(g, _):
            off = base + g * K
            pltpu.sync_copy(col_hbm.at[pl.ds(off, K)], colbuf)
            pltpu.sync_copy(ew_hbm.at[pl.ds(off, K)], ewbuf)
            pltpu.sync_copy(ewbuf, acc.at[colbuf], add=True)
            return ()

        lax.fori_loop(0, nch, body, (), unroll=False)
        plsc.subcore_barrier()

        @pl.when(s == 0)
        def _out():
            pltpu.sync_copy(acc, out_hbm.at[c])

    return k


def _msg_kernel(N, H, E):
    K = 50
    epw = E // (_NC * _NS)
    nch = epw // K
    rpw = (-(-N // _NS) + 7) // 8 * 8
    last = N - rpw * (_NS - 1)
    mesh = plsc.VectorSubcoreMesh(**_MESH)

    def _rows_copy(s, src_of, dst_of):
        @pl.when(s < _NS - 1)
        def _main():
            off = pl.multiple_of(s * rpw, rpw)
            pltpu.sync_copy(src_of(off, rpw), dst_of(off, rpw))

        @pl.when(s == _NS - 1)
        def _tail():
            pltpu.sync_copy(src_of(rpw * (_NS - 1), last),
                            dst_of(rpw * (_NS - 1), last))

    @functools.partial(
        pl.kernel,
        out_type=jax.ShapeDtypeStruct((_NC, N, H), jnp.float32),
        mesh=mesh,
        scratch_types=[
            pltpu.VMEM((epw,), jnp.int32),
            pltpu.VMEM((epw,), jnp.int32),
            pltpu.VMEM((epw,), jnp.float32),
            pltpu.VMEM((K, H), jnp.float32),
            pltpu.VMEM((K, H), jnp.float32),
            pltpu.VMEM((K,), jnp.int32),
            pltpu.VMEM((K,), jnp.int32),
            pltpu.VMEM((K,), jnp.int32),
            pltpu.VMEM((K,), jnp.int32),
            pltpu.VMEM((K,), jnp.float32),
            pltpu.VMEM((K,), jnp.float32),
            pltpu.VMEM_SHARED((N, H), jnp.float32),
            pltpu.SemaphoreType.DMA,
            pltpu.SemaphoreType.DMA,
            pltpu.SemaphoreType.DMA,
            pltpu.SemaphoreType.DMA,
        ],
    )
    def k(y_hbm, row_hbm, col_hbm, ew_hbm, zeros_hbm, out_hbm,
          rowi, coli, ewi, rows0, rows1, rstg0, rstg1, cstg0, cstg1,
          ews0, ews1, acc, sg0, sg1, ss0, ss1):
        c = lax.axis_index("c")
        s = lax.axis_index("s")
        rows = (rows0, rows1)
        rstg = (rstg0, rstg1)
        cstg = (cstg0, cstg1)
        ews = (ews0, ews1)
        sg = (sg0, sg1)
        ss = (ss0, ss1)

        _rows_copy(s, lambda o, l: zeros_hbm.at[pl.ds(o, l)],
                   lambda o, l: acc.at[pl.ds(o, l)])
        ebase = pl.multiple_of((c * _NS + s) * epw, 8)
        pltpu.sync_copy(row_hbm.at[pl.ds(ebase, epw)], rowi)
        pltpu.sync_copy(col_hbm.at[pl.ds(ebase, epw)], coli)
        pltpu.sync_copy(ew_hbm.at[pl.ds(ebase, epw)], ewi)
        plsc.subcore_barrier()

        def gather_start(b):
            pltpu.async_copy(y_hbm.at[rstg[b]], rows[b], sg[b])

        def gather_wait(b):
            pltpu.make_async_copy(y_hbm.at[rstg[b]], rows[b], sg[b]).wait()

        def scatter_start(b):
            pltpu.async_copy(rows[b], acc.at[cstg[b]], ss[b], add=True)

        def scatter_wait(b):
            pltpu.make_async_copy(rows[b], acc.at[cstg[b]], ss[b]).wait()

        def stage_idx(g, b):
            e = g * K
            for e0 in (0, 16, 32, 34):
                rstg[b][pl.ds(e0, 16)] = rowi[pl.ds(e + e0, 16)]
                cstg[b][pl.ds(e0, 16)] = coli[pl.ds(e + e0, 16)]
                ews[b][pl.ds(e0, 16)] = ewi[pl.ds(e + e0, 16)]

        def scale(b):
            for e0, lanes in ((0, range(16)), (16, range(16)),
                              (32, range(16)), (34, range(14, 16))):
                ew16 = ews[b][pl.ds(e0, 16)]
                for i in lanes:
                    e = e0 + i
                    if e0 == 34 and e < 48:
                        continue
                    w = ew16.at[jnp.full((16,), i, jnp.int32)].get(
                        mode="promise_in_bounds")
                    for q in range(H // 16):
                        sl = pl.ds(q * 16, 16)
                        rows[b][e, sl] = rows[b][e, sl] * w

        stage_idx(0, 0)
        gather_start(0)

        def body(gg, _):
            ga = 2 * gg
            gb = ga + 1

            @pl.when(gg > 0)
            def _drain_prev_odd():
                scatter_wait(1)

            stage_idx(gb, 1)
            gather_start(1)
            gather_wait(0)
            scale(0)
            scatter_start(0)

            @pl.when(gg < nch // 2 - 1)
            def _prefetch_next_even():
                scatter_wait(0)
                stage_idx(ga + 2, 0)
                gather_start(0)

            gather_wait(1)
            scale(1)
            scatter_start(1)
            return ()

        lax.fori_loop(0, nch // 2, body, (), unroll=False)
        scatter_wait(0)
        scatter_wait(1)
        plsc.subcore_barrier()
        _rows_copy(s, lambda o, l: acc.at[pl.ds(o, l)],
                   lambda o, l: out_hbm.at[c, pl.ds(o, l)])

    return k


def kernel(x, edge_attr, u, node_w, node_b, edge_w, edge_b, proj_w, proj_b,
           conv_w, conv_b, bn_g, bn_b, glob_w, glob_b, fin_w1, fin_b1,
           fin_w2, fin_b2, edge_index, batch):
    N, _ = x.shape
    H = conv_w.shape[1]
    E = edge_index.shape[1]
    L = conv_w.shape[0]

    row = edge_index[0]
    col = edge_index[1]

    ew = _edge_weights(edge_attr, edge_w, edge_b, proj_w, proj_b)

    zeros_n = jnp.zeros((N,), jnp.float32)
    zeros_nh = jnp.zeros((N, H), jnp.float32)

    deg = _deg_kernel(N, E)(col, ew, zeros_n).reshape(2, N, 1)
    batch2 = batch.reshape(N, 1)

    msg = _msg_kernel(N, H, E)

    y = _first_y(x, node_w, node_b, conv_w[0], deg)
    for i in range(L):
        s_parts = msg(y, row, col, ew, zeros_nh)
        if i + 1 < L:
            y = _next_y(s_parts, y, deg, conv_b[i], bn_g[i], bn_b[i],
                        conv_w[i + 1])
        else:
            out = _final(s_parts, y, deg, conv_b[i], bn_g[i], bn_b[i],
                         batch2, u, glob_w, glob_b, fin_w1, fin_b1,
                         fin_w2, fin_b2)
    return out


# speedup vs baseline: 1.0087x; 1.0087x over previous
import functools

import jax
import jax.numpy as jnp
from jax import lax
from jax.experimental import pallas as pl
from jax.experimental.pallas import tpu as pltpu
from jax.experimental.pallas import tpu_sc as plsc


def _lrelu(t):
    return jax.nn.leaky_relu(t, 0.01)


_BN = 2000
_BE = 16000


def _edge_mlp_body(attr_ref, ew_w_ref, ew_b_ref, pw_ref, pb_ref, out_ref):
    ea = _lrelu(
        jnp.dot(attr_ref[...], ew_w_ref[...], preferred_element_type=jnp.float32)
        + ew_b_ref[...]
    )
    t = jnp.dot(ea, pw_ref[...], preferred_element_type=jnp.float32) + pb_ref[...]
    ew = jax.nn.sigmoid(t)
    out_ref[...] = jnp.clip(ew, 1e-6, 1.0)


def _edge_weights(edge_attr, edge_w, edge_b, proj_w, proj_b):
    E = edge_attr.shape[0]
    grid = pl.cdiv(E, _BE)
    out = pl.pallas_call(
        _edge_mlp_body,
        grid=(grid,),
        in_specs=[
            pl.BlockSpec((_BE, edge_attr.shape[1]), lambda i: (i, 0)),
            pl.BlockSpec(edge_w.shape, lambda i: (0, 0)),
            pl.BlockSpec(edge_b.shape, lambda i: (0,)),
            pl.BlockSpec(proj_w.shape, lambda i: (0, 0)),
            pl.BlockSpec(proj_b.shape, lambda i: (0,)),
        ],
        out_specs=pl.BlockSpec((_BE, 1), lambda i: (i, 0)),
        out_shape=jax.ShapeDtypeStruct((E, 1), jnp.float32),
    )(edge_attr, edge_w, edge_b, proj_w, proj_b)
    return out.reshape(E)


def _first_y_body(x_ref, nw_ref, nb_ref, w0_ref, deg_ref, y_ref):
    h0 = _lrelu(
        jnp.dot(x_ref[...], nw_ref[...], preferred_element_type=jnp.float32)
        + nb_ref[...]
    )
    dinv = (deg_ref[:, 0] + 1.0) ** -0.5
    y_ref[...] = (
        jnp.dot(h0, w0_ref[...], preferred_element_type=jnp.float32)
        * dinv[:, None]
    )


def _first_y(x, node_w, node_b, conv_w0, deg):
    N, F = x.shape
    H = conv_w0.shape[1]
    return pl.pallas_call(
        _first_y_body,
        grid=(N // _BN,),
        in_specs=[
            pl.BlockSpec((_BN, F), lambda i: (i, 0)),
            pl.BlockSpec(node_w.shape, lambda i: (0, 0)),
            pl.BlockSpec(node_b.shape, lambda i: (0,)),
            pl.BlockSpec(conv_w0.shape, lambda i: (0, 0)),
            pl.BlockSpec((_BN, 1), lambda i: (i, 0)),
        ],
        out_specs=pl.BlockSpec((_BN, H), lambda i: (i, 0)),
        out_shape=jax.ShapeDtypeStruct((N, H), jnp.float32),
    )(x, node_w, node_b, conv_w0, deg)


def _next_y_body(s_ref, y_ref, deg_ref, cb_ref, g_ref, b_ref, w_ref, out_ref):
    dinv = (deg_ref[:, 0] + 1.0) ** -0.5
    s = s_ref[0] + s_ref[1]
    h = _lrelu(dinv[:, None] * (s + y_ref[...]) + cb_ref[...]) * g_ref[...] + b_ref[...]
    out_ref[...] = (
        jnp.dot(h, w_ref[...], preferred_element_type=jnp.float32) * dinv[:, None]
    )


def _next_y(s_parts, y_prev, deg, cb, g, b, w):
    N, H = y_prev.shape
    return pl.pallas_call(
        _next_y_body,
        grid=(N // _BN,),
        in_specs=[
            pl.BlockSpec((2, _BN, H), lambda i: (0, i, 0)),
            pl.BlockSpec((_BN, H), lambda i: (i, 0)),
            pl.BlockSpec((_BN, 1), lambda i: (i, 0)),
            pl.BlockSpec(cb.shape, lambda i: (0,)),
            pl.BlockSpec(g.shape, lambda i: (0,)),
            pl.BlockSpec(b.shape, lambda i: (0,)),
            pl.BlockSpec(w.shape, lambda i: (0, 0)),
        ],
        out_specs=pl.BlockSpec((_BN, H), lambda i: (i, 0)),
        out_shape=jax.ShapeDtypeStruct((N, H), jnp.float32),
    )(s_parts, y_prev, deg, cb, g, b, w)


def _final_body(s_ref, y_ref, deg_ref, cb_ref, g_ref, b_ref, batch_ref,
                u_ref, gw_ref, gb_ref, w1_ref, b1_ref, w2_ref, b2_ref,
                out_ref, pooled_acc, cnt_acc):
    j = pl.program_id(0)
    nb = pl.num_programs(0)

    @pl.when(j == 0)
    def _init():
        pooled_acc[...] = jnp.zeros_like(pooled_acc)
        cnt_acc[...] = jnp.zeros_like(cnt_acc)

    dinv = (deg_ref[:, 0] + 1.0) ** -0.5
    s = s_ref[0] + s_ref[1]
    h = _lrelu(dinv[:, None] * (s + y_ref[...]) + cb_ref[...]) * g_ref[...] + b_ref[...]

    B = pooled_acc.shape[0]
    onehot = (
        lax.broadcasted_iota(jnp.int32, (B, h.shape[0]), 0)
        == batch_ref[:, 0][None, :]
    ).astype(jnp.float32)
    pooled_acc[...] += jnp.dot(onehot, h, preferred_element_type=jnp.float32)
    cnt_acc[...] += jnp.sum(onehot, axis=1, keepdims=True)

    @pl.when(j == nb - 1)
    def _fin():
        pooled = pooled_acc[...] / jnp.maximum(cnt_acc[...], 1.0)
        ug = _lrelu(
            jnp.dot(u_ref[...], gw_ref[...], preferred_element_type=jnp.float32)
            + gb_ref[...]
        )
        comb = jnp.concatenate([pooled, ug], axis=1)
        hid = _lrelu(
            jnp.dot(comb, w1_ref[...], preferred_element_type=jnp.float32)
            + b1_ref[...]
        )
        out_ref[...] = (
            jnp.dot(hid, w2_ref[...], preferred_element_type=jnp.float32)
            + b2_ref[...]
        )


def _final(s_parts, y_prev, deg, cb, g, b, batch, u, glob_w, glob_b,
           fin_w1, fin_b1, fin_w2, fin_b2):
    N, H = y_prev.shape
    B = u.shape[0]
    return pl.pallas_call(
        _final_body,
        grid=(N // _BN,),
        in_specs=[
            pl.BlockSpec((2, _BN, H), lambda i: (0, i, 0)),
            pl.BlockSpec((_BN, H), lambda i: (i, 0)),
            pl.BlockSpec((_BN, 1), lambda i: (i, 0)),
            pl.BlockSpec(cb.shape, lambda i: (0,)),
            pl.BlockSpec(g.shape, lambda i: (0,)),
            pl.BlockSpec(b.shape, lambda i: (0,)),
            pl.BlockSpec((_BN, 1), lambda i: (i, 0)),
            pl.BlockSpec(u.shape, lambda i: (0, 0)),
            pl.BlockSpec(glob_w.shape, lambda i: (0, 0)),
            pl.BlockSpec(glob_b.shape, lambda i: (0,)),
            pl.BlockSpec(fin_w1.shape, lambda i: (0, 0)),
            pl.BlockSpec(fin_b1.shape, lambda i: (0,)),
            pl.BlockSpec(fin_w2.shape, lambda i: (0, 0)),
            pl.BlockSpec(fin_b2.shape, lambda i: (0,)),
        ],
        out_specs=pl.BlockSpec((B, 2), lambda i: (0, 0)),
        out_shape=jax.ShapeDtypeStruct((B, 2), jnp.float32),
        scratch_shapes=[
            pltpu.VMEM((B, H), jnp.float32),
            pltpu.VMEM((B, 1), jnp.float32),
        ],
    )(s_parts, y_prev, deg, cb, g, b, batch, u, glob_w, glob_b,
      fin_w1, fin_b1, fin_w2, fin_b2)


_NC = 2
_NS = 16
_MESH = dict(core_axis_name="c", subcore_axis_name="s",
             num_cores=_NC, num_subcores=_NS)


def _deg_kernel(N, E):
    epw = E // (_NC * _NS)
    K = 1000
    nch = epw // K
    mesh = plsc.VectorSubcoreMesh(**_MESH)

    @functools.partial(
        pl.kernel,
        out_type=jax.ShapeDtypeStruct((_NC, N), jnp.float32),
        mesh=mesh,
        scratch_types=[
            pltpu.VMEM((K,), jnp.int32),
            pltpu.VMEM((K,), jnp.float32),
            pltpu.VMEM_SHARED((N,), jnp.float32),
        ],
    )
    def k(col_hbm, ew_hbm, zeros_hbm, out_hbm, colbuf, ewbuf, acc):
        c = lax.axis_index("c")
        s = lax.axis_index("s")

        @pl.when(s == 0)
        def _zero():
            pltpu.sync_copy(zeros_hbm, acc)

        plsc.subcore_barrier()
        base = (c * _NS + s) * epw

        def body(g, _):
            off = base + g * K
            pltpu.sync_copy(col_hbm.at[pl.ds(off, K)], colbuf)
            pltpu.sync_copy(ew_hbm.at[pl.ds(off, K)], ewbuf)
            pltpu.sync_copy(ewbuf, acc.at[colbuf], add=True)
            return ()

        lax.fori_loop(0, nch, body, (), unroll=False)
        plsc.subcore_barrier()

        @pl.when(s == 0)
        def _out():
            pltpu.sync_copy(acc, out_hbm.at[c])

    return k


def _msg_kernel(N, H, E):
    K = 50
    epw = E // (_NC * _NS)
    nch = epw // K
    rpw = (-(-N // _NS) + 7) // 8 * 8
    last = N - rpw * (_NS - 1)
    mesh = plsc.VectorSubcoreMesh(**_MESH)

    def _rows_copy(s, src_of, dst_of):
        @pl.when(s < _NS - 1)
        def _main():
            off = pl.multiple_of(s * rpw, rpw)
            pltpu.sync_copy(src_of(off, rpw), dst_of(off, rpw))

        @pl.when(s == _NS - 1)
        def _tail():
            pltpu.sync_copy(src_of(rpw * (_NS - 1), last),
                            dst_of(rpw * (_NS - 1), last))

    @functools.partial(
        pl.kernel,
        out_type=jax.ShapeDtypeStruct((_NC, N, H), jnp.float32),
        mesh=mesh,
        scratch_types=[
            pltpu.VMEM((epw,), jnp.int32),
            pltpu.VMEM((epw,), jnp.int32),
            pltpu.VMEM((epw,), jnp.float32),
            pltpu.VMEM((K, H), jnp.float32),
            pltpu.VMEM((K, H), jnp.float32),
            pltpu.VMEM((K,), jnp.int32),
            pltpu.VMEM((K,), jnp.int32),
            pltpu.VMEM((K,), jnp.int32),
            pltpu.VMEM((K,), jnp.int32),
            pltpu.VMEM((K,), jnp.float32),
            pltpu.VMEM((K,), jnp.float32),
            pltpu.VMEM_SHARED((N, H), jnp.float32),
            pltpu.SemaphoreType.DMA,
            pltpu.SemaphoreType.DMA,
            pltpu.SemaphoreType.DMA,
            pltpu.SemaphoreType.DMA,
        ],
    )
    def k(y_hbm, row_hbm, col_hbm, ew_hbm, zeros_hbm, out_hbm,
          rowi, coli, ewi, rows0, rows1, rstg0, rstg1, cstg0, cstg1,
          ews0, ews1, acc, sg0, sg1, ss0, ss1):
        c = lax.axis_index("c")
        s = lax.axis_index("s")
        rows = (rows0, rows1)
        rstg = (rstg0, rstg1)
        cstg = (cstg0, cstg1)
        ews = (ews0, ews1)
        sg = (sg0, sg1)
        ss = (ss0, ss1)

        _rows_copy(s, lambda o, l: zeros_hbm.at[pl.ds(o, l)],
                   lambda o, l: acc.at[pl.ds(o, l)])
        ebase = pl.multiple_of((c * _NS + s) * epw, 8)
        pltpu.sync_copy(row_hbm.at[pl.ds(ebase, epw)], rowi)
        pltpu.sync_copy(col_hbm.at[pl.ds(ebase, epw)], coli)
        pltpu.sync_copy(ew_hbm.at[pl.ds(ebase, epw)], ewi)
        plsc.subcore_barrier()

        def gather_start(b):
            pltpu.async_copy(y_hbm.at[rstg[b]], rows[b], sg[b])

        def gather_wait(b):
            pltpu.make_async_copy(y_hbm.at[rstg[b]], rows[b], sg[b]).wait()

        def scatter_start(b):
            pltpu.async_copy(rows[b], acc.at[cstg[b]], ss[b], add=True)

        def scatter_wait(b):
            pltpu.make_async_copy(rows[b], acc.at[cstg[b]], ss[b]).wait()

        def stage_idx(g, b):
            e = g * K
            for e0 in (0, 16, 32, 34):
                rstg[b][pl.ds(e0, 16)] = rowi[pl.ds(e + e0, 16)]
                cstg[b][pl.ds(e0, 16)] = coli[pl.ds(e + e0, 16)]
                ews[b][pl.ds(e0, 16)] = ewi[pl.ds(e + e0, 16)]

        def scale(b):
            for e0, lanes in ((0, range(16)), (16, range(16)),
                              (32, range(16)), (34, range(14, 16))):
                ew16 = ews[b][pl.ds(e0, 16)]
                for i in lanes:
                    e = e0 + i
                    if e0 == 34 and e < 48:
                        continue
                    w = ew16.at[jnp.full((16,), i, jnp.int32)].get(
                        mode="promise_in_bounds")
                    for q in range(H // 16):
                        sl = pl.ds(q * 16, 16)
                        rows[b][e, sl] = rows[b][e, sl] * w

        stage_idx(0, 0)
        gather_start(0)

        def body(gg, _):
            ga = 2 * gg
            gb = ga + 1

            @pl.when(gg > 0)
            def _drain_prev_odd():
                scatter_wait(1)

            stage_idx(gb, 1)
            gather_start(1)
            gather_wait(0)
            scale(0)
            scatter_start(0)

            @pl.when(gg < nch // 2 - 1)
            def _prefetch_next_even():
                scatter_wait(0)
                stage_idx(ga + 2, 0)
                gather_start(0)

            gather_wait(1)
            scale(1)
            scatter_start(1)
            return ()

        lax.fori_loop(0, nch // 2, body, (), unroll=False)
        scatter_wait(0)
        scatter_wait(1)
        plsc.subcore_barrier()
        _rows_copy(s, lambda o, l: acc.at[pl.ds(o, l)],
                   lambda o, l: out_hbm.at[c, pl.ds(o, l)])

    return k


def kernel(x, edge_attr, u, node_w, node_b, edge_w, edge_b, proj_w, proj_b,
           conv_w, conv_b, bn_g, bn_b, glob_w, glob_b, fin_w1, fin_b1,
           fin_w2, fin_b2, edge_index, batch):
    N, _ = x.shape
    H = conv_w.shape[1]
    E = edge_index.shape[1]
    L = conv_w.shape[0]

    row = edge_index[0]
    col = edge_index[1]

    ew = _edge_weights(edge_attr, edge_w, edge_b, proj_w, proj_b)

    zeros_n = jnp.zeros((N,), jnp.float32)
    zeros_nh = jnp.zeros((N, H), jnp.float32)

    deg_parts = _deg_kernel(N, E)(col, ew, zeros_n)
    deg = (deg_parts[0] + deg_parts[1]).reshape(N, 1)
    batch2 = batch.reshape(N, 1)

    msg = _msg_kernel(N, H, E)

    y = _first_y(x, node_w, node_b, conv_w[0], deg)
    for i in range(L):
        s_parts = msg(y, row, col, ew, zeros_nh)
        if i + 1 < L:
            y = _next_y(s_parts, y, deg, conv_b[i], bn_g[i], bn_b[i],
                        conv_w[i + 1])
        else:
            out = _final(s_parts, y, deg, conv_b[i], bn_g[i], bn_b[i],
                         batch2, u, glob_w, glob_b, fin_w1, fin_b1,
                         fin_w2, fin_b2)
    return out
